# baseline (speedup 1.0000x reference)
import jax
import jax.numpy as jnp
from jax.experimental import pallas as pl
from jax.experimental.pallas import tpu as pltpu

N = 10000
NFEAT = 128
NHID = 128
NCLASS = 16
SHID = N // 3

BV = 200
NGRP = 10
BW1 = 256
NB1 = 10
BH = 384
NB2 = pl.cdiv(SHID, BH)
BM = 400
NBM = N // BM


def _a0_kernel(xv_ref, s_ref):
    xv = xv_ref[...]
    parts = [jnp.sum(xv[:, 128 * b:128 * (b + 1)], axis=1, keepdims=True)
             for b in range(NGRP)]
    s_ref[...] = jnp.concatenate(parts, axis=1)


def _a1_kernel(w10_ref, w11_ref, w12_ref, w13_ref,
               s0_ref, s1_ref, s2_ref, s3_ref, b1_ref, h_ref, hacc_ref):
    i = pl.program_id(0)
    sub = jax.lax.broadcasted_iota(jnp.int32, (BW1, 1), 0)

    def _masked(w_ref, s_ref, k):
        grow = BW1 * (NB1 * k + i) + sub
        return jnp.sum(jnp.where(grow < N, w_ref[...] * s_ref[...], 0.0),
                       axis=0, keepdims=True)

    part = (_masked(w10_ref, s0_ref, 0) + _masked(w11_ref, s1_ref, 1)
            + _masked(w12_ref, s2_ref, 2) + _masked(w13_ref, s3_ref, 3))

    @pl.when(i == 0)
    def _():
        hacc_ref[...] = part

    @pl.when(i > 0)
    def _():
        hacc_ref[...] = hacc_ref[...] + part

    @pl.when(i == NB1 - 1)
    def _():
        h_ref[...] = jax.nn.relu(hacc_ref[...] + b1_ref[...])


def _a2_kernel(h_ref, w2_ref, b2_ref, sc_ref):
    i = pl.program_id(0)
    row = jax.lax.broadcasted_iota(jnp.int32, (BH, 1), 0) + i * BH
    prod = jnp.where(row < SHID, w2_ref[...] * h_ref[...], 0.0)
    part = jnp.sum(prod, axis=0, keepdims=True)

    @pl.when(i == 0)
    def _():
        sc_ref[...] = part

    @pl.when(i > 0)
    def _():
        sc_ref[...] = sc_ref[...] + part

    @pl.when(i == NB2 - 1)
    def _():
        sc_ref[...] = jax.nn.sigmoid(sc_ref[...] + b2_ref[...])


def _bc_kernel(adj_ref, x_ref, wg_ref, bg_ref, wcat_ref, score_ref, eps_ref,
               b11_ref, b12_ref, out_ref, m_ref):
    i = pl.program_id(0)

    @pl.when(i < NBM)
    def _():
        t = jax.lax.dot_general(
            adj_ref[...], x_ref[...], (((1,), (0,)), ((), ())),
            preferred_element_type=jnp.float32)
        h = jax.lax.dot_general(
            t, wg_ref[...], (((1,), (0,)), ((), ())),
            preferred_element_type=jnp.float32)
        h = jax.nn.relu(h + bg_ref[...])
        hw = jax.lax.dot_general(
            h, wcat_ref[...], (((1,), (0,)), ((), ())),
            preferred_element_type=jnp.float32)
        m_ref[pl.ds(i * BM, BM), :] = hw * score_ref[pl.ds(i * BM, BM), :]

    @pl.when(i >= NBM)
    def _():
        acc = jax.lax.dot_general(
            adj_ref[...], m_ref[...], (((1,), (0,)), ((), ())),
            preferred_element_type=jnp.float32)
        mean = acc[:, :NCLASS] + b11_ref[...]
        logstd = acc[:, NCLASS:] + b12_ref[...]
        z = eps_ref[...] * jnp.exp(logstd) + mean
        zmax = jnp.max(z, axis=1, keepdims=True)
        ze = z - zmax
        out_ref[...] = ze - jnp.log(
            jnp.sum(jnp.exp(ze), axis=1, keepdims=True))


def kernel(x, adj, W_gc1, b_gc1, W_fc11, b_fc11, W_fc12, b_fc12,
           W_se1, b_se1, W_se2, b_se2, eps):
    f32 = jnp.float32

    xv = x.reshape(N // NGRP, NGRP * NFEAT)

    s_ab = pl.pallas_call(
        _a0_kernel,
        grid=(N // NGRP // BV,),
        in_specs=[pl.BlockSpec((BV, NGRP * NFEAT), lambda i: (i, 0))],
        out_specs=pl.BlockSpec((BV, NGRP), lambda i: (i, 0)),
        out_shape=jax.ShapeDtypeStruct((N // NGRP, NGRP), f32),
        compiler_params=pltpu.CompilerParams(
            dimension_semantics=("parallel",)),
    )(xv)

    s_col = s_ab.reshape(N, 1)

    w1_specs = [
        pl.BlockSpec((BW1, SHID),
                     (lambda k: (lambda i: (NB1 * k + i, 0)))(k))
        for k in range(4)
    ]
    s_specs = [
        pl.BlockSpec((BW1, 1),
                     (lambda k: (lambda i: (NB1 * k + i, 0)))(k))
        for k in range(4)
    ]

    h = pl.pallas_call(
        _a1_kernel,
        grid=(NB1,),
        in_specs=w1_specs + s_specs + [
            pl.BlockSpec((1, SHID), lambda i: (0, 0))],
        out_specs=pl.BlockSpec((1, SHID), lambda i: (0, 0)),
        out_shape=jax.ShapeDtypeStruct((1, SHID), f32),
        scratch_shapes=[pltpu.VMEM((1, SHID), f32)],
        compiler_params=pltpu.CompilerParams(
            dimension_semantics=("arbitrary",)),
    )(W_se1, W_se1, W_se1, W_se1, s_col, s_col, s_col, s_col,
      b_se1.reshape(1, SHID))

    h_col = h.reshape(SHID, 1)

    sc_row = pl.pallas_call(
        _a2_kernel,
        grid=(NB2,),
        in_specs=[
            pl.BlockSpec((BH, 1), lambda i: (i, 0)),
            pl.BlockSpec((BH, N), lambda i: (i, 0)),
            pl.BlockSpec((1, N), lambda i: (0, 0)),
        ],
        out_specs=pl.BlockSpec((1, N), lambda i: (0, 0)),
        out_shape=jax.ShapeDtypeStruct((1, N), f32),
        compiler_params=pltpu.CompilerParams(
            dimension_semantics=("arbitrary",)),
    )(h_col, W_se2, b_se2.reshape(1, N))

    score = sc_row.reshape(N, 1)
    wcat = jnp.concatenate([W_fc11, W_fc12], axis=1)

    out = pl.pallas_call(
        _bc_kernel,
        grid=(2 * NBM,),
        in_specs=[
            pl.BlockSpec((BM, N), lambda i: (jax.lax.rem(i, NBM), 0)),
            pl.BlockSpec((N, NFEAT), lambda i: (0, 0)),
            pl.BlockSpec((NFEAT, NHID), lambda i: (0, 0)),
            pl.BlockSpec((1, NHID), lambda i: (0, 0)),
            pl.BlockSpec((NHID, 2 * NCLASS), lambda i: (0, 0)),
            pl.BlockSpec((N, 1), lambda i: (0, 0)),
            pl.BlockSpec((BM, NCLASS), lambda i: (jax.lax.rem(i, NBM), 0)),
            pl.BlockSpec((1, NCLASS), lambda i: (0, 0)),
            pl.BlockSpec((1, NCLASS), lambda i: (0, 0)),
        ],
        out_specs=pl.BlockSpec((BM, NCLASS),
                               lambda i: (jax.lax.rem(i, NBM), 0)),
        out_shape=jax.ShapeDtypeStruct((N, NCLASS), f32),
        scratch_shapes=[pltpu.VMEM((N, 2 * NCLASS), f32)],
        compiler_params=pltpu.CompilerParams(
            dimension_semantics=("arbitrary",)),
    )(adj, x, W_gc1, b_gc1.reshape(1, NHID), wcat, score, eps,
      b_fc11.reshape(1, NCLASS), b_fc12.reshape(1, NCLASS))

    return out


# speedup vs baseline: 1.0485x; 1.0485x over previous
import jax
import jax.numpy as jnp
from jax.experimental import pallas as pl
from jax.experimental.pallas import tpu as pltpu

N = 10000
NFEAT = 128
NHID = 128
NCLASS = 16
SHID = N // 3

BX = 1000
NB1 = N // BX
BH = 384
NB2 = pl.cdiv(SHID, BH)
BM = 400
NBM = N // BM


def _a1_kernel(x_ref, wg_ref, w1_ref, b1_ref, xw_ref, h_ref, hacc_ref):
    i = pl.program_id(0)
    x = x_ref[...]
    xw_ref[...] = jax.lax.dot_general(
        x, wg_ref[...], (((1,), (0,)), ((), ())),
        preferred_element_type=jnp.float32)
    s = jnp.sum(x, axis=1, keepdims=True)
    part = jnp.sum(w1_ref[...] * s, axis=0, keepdims=True)

    @pl.when(i == 0)
    def _():
        hacc_ref[...] = part

    @pl.when(i > 0)
    def _():
        hacc_ref[...] = hacc_ref[...] + part

    @pl.when(i == NB1 - 1)
    def _():
        h_ref[...] = jax.nn.relu(hacc_ref[...] + b1_ref[...])


def _a2_kernel(h_ref, w2_ref, b2_ref, sc_ref):
    i = pl.program_id(0)
    hc = h_ref[...].reshape(BH, 1)
    row = jax.lax.broadcasted_iota(jnp.int32, (BH, 1), 0) + i * BH
    prod = jnp.where(row < SHID, w2_ref[...] * hc, 0.0)
    part = jnp.sum(prod, axis=0, keepdims=True)

    @pl.when(i == 0)
    def _():
        sc_ref[...] = part

    @pl.when(i > 0)
    def _():
        sc_ref[...] = sc_ref[...] + part

    @pl.when(i == NB2 - 1)
    def _():
        sc_ref[...] = jax.nn.sigmoid(sc_ref[...] + b2_ref[...])


def _bc_kernel(adj_ref, xw_ref, bg_ref, wcat_ref, score_ref, eps_ref,
               b11_ref, b12_ref, out_ref, m_ref):
    i = pl.program_id(0)

    @pl.when(i < NBM)
    def _():
        h = jax.lax.dot_general(
            adj_ref[...], xw_ref[...], (((1,), (0,)), ((), ())),
            preferred_element_type=jnp.float32)
        h = jax.nn.relu(h + bg_ref[...])
        hw = jax.lax.dot_general(
            h, wcat_ref[...], (((1,), (0,)), ((), ())),
            preferred_element_type=jnp.float32)
        m_ref[pl.ds(i * BM, BM), :] = hw

    @pl.when(i >= NBM)
    def _():
        adj_s = adj_ref[...] * score_ref[...]
        acc = jax.lax.dot_general(
            adj_s, m_ref[...], (((1,), (0,)), ((), ())),
            preferred_element_type=jnp.float32)
        mean = acc[:, :NCLASS] + b11_ref[...]
        logstd = acc[:, NCLASS:] + b12_ref[...]
        z = eps_ref[...] * jnp.exp(logstd) + mean
        zmax = jnp.max(z, axis=1, keepdims=True)
        ze = z - zmax
        out_ref[...] = ze - jnp.log(
            jnp.sum(jnp.exp(ze), axis=1, keepdims=True))


def kernel(x, adj, W_gc1, b_gc1, W_fc11, b_fc11, W_fc12, b_fc12,
           W_se1, b_se1, W_se2, b_se2, eps):
    f32 = jnp.float32

    xw, h = pl.pallas_call(
        _a1_kernel,
        grid=(NB1,),
        in_specs=[
            pl.BlockSpec((BX, NFEAT), lambda i: (i, 0)),
            pl.BlockSpec((NFEAT, NHID), lambda i: (0, 0)),
            pl.BlockSpec((BX, SHID), lambda i: (i, 0)),
            pl.BlockSpec((1, SHID), lambda i: (0, 0)),
        ],
        out_specs=[
            pl.BlockSpec((BX, NHID), lambda i: (i, 0)),
            pl.BlockSpec((1, SHID), lambda i: (0, 0)),
        ],
        out_shape=[
            jax.ShapeDtypeStruct((N, NHID), f32),
            jax.ShapeDtypeStruct((1, SHID), f32),
        ],
        scratch_shapes=[pltpu.VMEM((1, SHID), f32)],
        compiler_params=pltpu.CompilerParams(
            dimension_semantics=("arbitrary",)),
    )(x, W_gc1, W_se1, b_se1.reshape(1, SHID))

    sc_row = pl.pallas_call(
        _a2_kernel,
        grid=(NB2,),
        in_specs=[
            pl.BlockSpec((1, BH), lambda i: (0, i)),
            pl.BlockSpec((BH, N), lambda i: (i, 0)),
            pl.BlockSpec((1, N), lambda i: (0, 0)),
        ],
        out_specs=pl.BlockSpec((1, N), lambda i: (0, 0)),
        out_shape=jax.ShapeDtypeStruct((1, N), f32),
        compiler_params=pltpu.CompilerParams(
            dimension_semantics=("arbitrary",)),
    )(h, W_se2, b_se2.reshape(1, N))

    wcat = jnp.concatenate([W_fc11, W_fc12], axis=1)

    out = pl.pallas_call(
        _bc_kernel,
        grid=(2 * NBM,),
        in_specs=[
            pl.BlockSpec((BM, N), lambda i: (jax.lax.rem(i, NBM), 0)),
            pl.BlockSpec((N, NHID), lambda i: (0, 0)),
            pl.BlockSpec((1, NHID), lambda i: (0, 0)),
            pl.BlockSpec((NHID, 2 * NCLASS), lambda i: (0, 0)),
            pl.BlockSpec((1, N), lambda i: (0, 0)),
            pl.BlockSpec((BM, NCLASS), lambda i: (jax.lax.rem(i, NBM), 0)),
            pl.BlockSpec((1, NCLASS), lambda i: (0, 0)),
            pl.BlockSpec((1, NCLASS), lambda i: (0, 0)),
        ],
        out_specs=pl.BlockSpec((BM, NCLASS),
                               lambda i: (jax.lax.rem(i, NBM), 0)),
        out_shape=jax.ShapeDtypeStruct((N, NCLASS), f32),
        scratch_shapes=[pltpu.VMEM((N, 2 * NCLASS), f32)],
        compiler_params=pltpu.CompilerParams(
            dimension_semantics=("arbitrary",)),
    )(adj, xw, b_gc1.reshape(1, NHID), wcat, sc_row, eps,
      b_fc11.reshape(1, NCLASS), b_fc12.reshape(1, NCLASS))

    return out
